# Initial kernel scaffold; baseline (speedup 1.0000x reference)
import jax
import jax.numpy as jnp
from jax.experimental import pallas as pl


def kernel(transformation_matrices, pred, true):
    raise NotImplementedError("write your pallas kernel here")


# speedup vs baseline: 6.2707x; 6.2707x over previous
import functools

import jax
import jax.numpy as jnp
from jax import lax
from jax.experimental import pallas as pl
from jax.experimental.pallas import tpu as pltpu
from jax.experimental.pallas import tpu_sc as plsc

NROWS = 128
NCOLS = 32768
TOPK = 256
NC = 2
NS = 16
L = 16
NW = NC * NS
ROWS_PER_W = NROWS // NW
CHUNK = 8192
NCHUNKS = NCOLS // CHUNK
VPC = CHUNK // L
NBINS = 256


def _key(p):
    p = jnp.where(p == 0.0, jnp.float32(0.0), p)
    s = lax.bitcast_convert_type(p, jnp.int32)
    return jnp.where(s >= 0, s, s ^ jnp.int32(0x7FFFFFFF))


def _zero_hist(hist):
    z = jnp.zeros((L,), jnp.int32)

    def zh(i, _):
        hist[pl.ds(i * L, L)] = z
        return 0

    lax.fori_loop(0, NBINS * L // L, zh, 0)


def _find_threshold(hist, lane, kneed):
    def step(vi, carry):
        run, bsel, cgt = carry
        v = 15 - vi

        def ml(l, acc):
            return acc + hist[pl.ds(l * NBINS + v * L, L)]

        mvec = lax.fori_loop(0, L, ml, jnp.zeros((L,), jnp.int32))
        rev = lax.rev(mvec, (0,))
        cum = plsc.cumsum(rev)
        sfx = lax.rev(cum - rev, (0,)) + run
        sel = jnp.logical_and(sfx < kneed, sfx + mvec >= kneed)
        bins = v * L + lane
        bsel = jnp.maximum(bsel, jnp.max(jnp.where(sel, bins, -1)))
        cgt = jnp.maximum(cgt, jnp.max(jnp.where(sel, sfx, -1)))
        return run + jnp.sum(mvec), bsel, cgt

    _, bsel, cgt = lax.fori_loop(
        0, L, step, (jnp.int32(0), jnp.int32(-1), jnp.int32(-1))
    )
    return bsel, cgt


def _body(pred_hbm, true_hbm, out_hbm, pch, tch, hist, candk, candd, outv):
    wid = lax.axis_index("s") * NC + lax.axis_index("c")
    lane = lax.iota(jnp.int32, L)
    ones_i = jnp.ones((L,), jnp.int32)

    def round_fn(ridx, carry):
        m, kneed, accum = carry
        shift = 16 - 8 * ridx
        _zero_hist(hist)
        nvec = (m + L - 1) // L

        def hstep(j, _):
            kk = candk[pl.ds(j * L, L)]
            valid = (j * L + lane) < m
            digit = jnp.right_shift(kk, shift) & 255
            plsc.addupdate_scatter(hist, [lane * NBINS + digit], ones_i,
                                   mask=valid)
            return 0

        lax.fori_loop(0, nvec, hstep, 0)
        bd, cgt = _find_threshold(hist, lane, kneed)

        def sstep(j, carry2):
            acc, off2 = carry2
            kk = candk[pl.ds(j * L, L)]
            dd = candd[pl.ds(j * L, L)]
            valid = (j * L + lane) < m
            digit = jnp.right_shift(kk, shift) & 255
            gt = jnp.logical_and(valid, digit > bd)
            acc = acc + jnp.where(gt, dd, 0.0)
            eq = jnp.logical_and(valid, digit == bd)
            plsc.store_compressed(candk.at[pl.ds(off2, L)], kk, mask=eq)
            plsc.store_compressed(candd.at[pl.ds(off2, L)], dd, mask=eq)
            popc = plsc.all_reduce_population_count(eq)
            return acc, off2 + jnp.max(popc)

        accum, m2 = lax.fori_loop(0, nvec, sstep, (accum, jnp.int32(0)))
        return m2, kneed - cgt, accum

    def row_fn(r, total):
        rowbase = (wid * ROWS_PER_W + r) * NCOLS
        _zero_hist(hist)

        def chunk_a(c, _):
            pltpu.sync_copy(pred_hbm.at[pl.ds(rowbase + c * CHUNK, CHUNK)],
                            pch)

            def vec_a(j, _):
                kk = _key(pch[pl.ds(j * L, L)])
                bin8 = jnp.right_shift(kk, 24) + 128
                plsc.addupdate_scatter(hist, [lane * NBINS + bin8], ones_i)
                return 0

            lax.fori_loop(0, VPC, vec_a, 0)
            return 0

        lax.fori_loop(0, NCHUNKS, chunk_a, 0)
        b1, c1 = _find_threshold(hist, lane, jnp.int32(TOPK))

        def chunk_b(c, carry):
            accum, off = carry
            base = rowbase + c * CHUNK
            pltpu.sync_copy(pred_hbm.at[pl.ds(base, CHUNK)], pch)
            pltpu.sync_copy(true_hbm.at[pl.ds(base, CHUNK)], tch)

            def vec_b(j, c2):
                accum, off = c2
                p = pch[pl.ds(j * L, L)]
                t = tch[pl.ds(j * L, L)]
                kk = _key(p)
                bin8 = jnp.right_shift(kk, 24) + 128
                d = p - t
                d2 = d * d
                accum = accum + jnp.where(bin8 > b1, d2, 0.0)
                eq = bin8 == b1
                plsc.store_compressed(candk.at[pl.ds(off, L)], kk, mask=eq)
                plsc.store_compressed(candd.at[pl.ds(off, L)], d2, mask=eq)
                popc = plsc.all_reduce_population_count(eq)
                return accum, off + jnp.max(popc)

            return lax.fori_loop(0, VPC, vec_b, (accum, off))

        accum, m = lax.fori_loop(
            0, NCHUNKS, chunk_b,
            (jnp.zeros((L,), jnp.float32), jnp.int32(0)),
        )

        m, kneed, accum = lax.fori_loop(0, 3, round_fn,
                                        (m, TOPK - c1, accum))

        nv = (kneed + L - 1) // L

        def fstep(j, acc):
            dd = candd[pl.ds(j * L, L)]
            sel = (j * L + lane) < kneed
            return acc + jnp.where(sel, dd, 0.0)

        accum = lax.fori_loop(0, nv, fstep, accum)
        return total + jnp.sum(accum)

    total = lax.fori_loop(0, ROWS_PER_W, row_fn, jnp.float32(0.0))
    outv[...] = jnp.where(lane == 0, total * (1.0 / (TOPK * NROWS)), 0.0)
    pltpu.sync_copy(outv, out_hbm.at[wid])


@jax.jit
def _topk_mse(pred_flat, true_flat):
    mesh = plsc.VectorSubcoreMesh(
        core_axis_name="c", subcore_axis_name="s", num_cores=NC,
        num_subcores=NS,
    )
    return pl.kernel(
        _body,
        out_type=jax.ShapeDtypeStruct((NW, L), jnp.float32),
        mesh=mesh,
        compiler_params=pltpu.CompilerParams(needs_layout_passes=False),
        scratch_types=[
            pltpu.VMEM((CHUNK,), jnp.float32),
            pltpu.VMEM((CHUNK,), jnp.float32),
            pltpu.VMEM((NBINS * L,), jnp.int32),
            pltpu.VMEM((NCOLS + L,), jnp.int32),
            pltpu.VMEM((NCOLS + L,), jnp.float32),
            pltpu.VMEM((L,), jnp.float32),
        ],
    )(pred_flat, true_flat)


def kernel(transformation_matrices, pred, true):
    del transformation_matrices
    pred = jnp.squeeze(pred)
    true = jnp.squeeze(true)
    partials = _topk_mse(pred.reshape(-1), true.reshape(-1))
    return jnp.sum(partials)


# speedup vs baseline: 6.5669x; 1.0472x over previous
import jax
import jax.numpy as jnp
from jax import lax
from jax.experimental import pallas as pl
from jax.experimental.pallas import tpu as pltpu
from jax.experimental.pallas import tpu_sc as plsc

NROWS = 128
NCOLS = 32768
TOPK = 256
NC = 2
NS = 16
L = 16
NW = NC * NS
ROWS_PER_W = NROWS // NW
CHUNK = 8192
NCHUNKS = NCOLS // CHUNK
VPC = CHUNK // L
NBINS = 256


def _ukey(p):
    p = jnp.where(p == 0.0, jnp.float32(0.0), p)
    s = lax.bitcast_convert_type(p, jnp.int32)
    return jnp.where(s >= 0, s, s ^ jnp.int32(0x7FFFFFFF)) ^ jnp.int32(
        -(2**31))


def _zero_hist(hist):
    z = jnp.zeros((L,), jnp.int32)

    def zh(i, _):
        hist[pl.ds(i * L, L)] = z
        return 0

    lax.fori_loop(0, NBINS * L // L, zh, 0)


def _find_threshold(hist, lane, kneed):
    def step(vi, carry):
        run, bsel, cgt = carry
        v = 15 - vi

        def ml(l, acc):
            return acc + hist[pl.ds(l * NBINS + v * L, L)]

        mvec = lax.fori_loop(0, L, ml, jnp.zeros((L,), jnp.int32))
        rev = lax.rev(mvec, (0,))
        cum = plsc.cumsum(rev)
        sfx = lax.rev(cum - rev, (0,)) + run
        sel = jnp.logical_and(sfx < kneed, sfx + mvec >= kneed)
        bins = v * L + lane
        bsel = jnp.maximum(bsel, jnp.max(jnp.where(sel, bins, -1)))
        cgt = jnp.maximum(cgt, jnp.max(jnp.where(sel, sfx, -1)))
        return run + jnp.sum(mvec), bsel, cgt

    _, bsel, cgt = lax.fori_loop(
        0, L, step, (jnp.int32(0), jnp.int32(-1), jnp.int32(-1))
    )
    return bsel, cgt


def _body(pred_hbm, true_hbm, out_hbm, pch, tch, hist, keybuf, d2buf, outv):
    wid = lax.axis_index("s") * NC + lax.axis_index("c")
    lane = lax.iota(jnp.int32, L)
    ones_i = jnp.ones((L,), jnp.int32)

    def radix_round(shift, m, kneed, accum, build_hist):
        nvec = (m + L - 1) // L
        if build_hist:
            _zero_hist(hist)

            def hstep(j, _):
                kk = keybuf[pl.ds(j * L, L)]
                valid = (j * L + lane) < m
                digit = jnp.right_shift(kk, shift) & 255
                plsc.addupdate_scatter(hist, [lane * NBINS + digit], ones_i,
                                       mask=valid)
                return 0

            lax.fori_loop(0, nvec, hstep, 0)
        bd, cgt = _find_threshold(hist, lane, kneed)

        def sstep(j, carry):
            acc, off = carry
            kk = keybuf[pl.ds(j * L, L)]
            dd = d2buf[pl.ds(j * L, L)]
            valid = (j * L + lane) < m
            digit = jnp.right_shift(kk, shift) & 255
            gt = jnp.logical_and(valid, digit > bd)
            acc = acc + jnp.where(gt, dd, 0.0)
            eq = jnp.logical_and(valid, digit == bd)
            plsc.store_compressed(keybuf.at[pl.ds(off, L)], kk, mask=eq)
            plsc.store_compressed(d2buf.at[pl.ds(off, L)], dd, mask=eq)
            popc = plsc.all_reduce_population_count(eq)
            return acc, off + jnp.max(popc)

        accum, m2 = lax.fori_loop(0, nvec, sstep, (accum, jnp.int32(0)))
        return m2, kneed - cgt, accum

    def row_fn(r, total):
        rowbase = (wid * ROWS_PER_W + r) * NCOLS
        _zero_hist(hist)

        def chunk_a(c, _):
            base = rowbase + c * CHUNK
            pltpu.sync_copy(pred_hbm.at[pl.ds(base, CHUNK)], pch)
            pltpu.sync_copy(true_hbm.at[pl.ds(base, CHUNK)], tch)

            def vec_a(j, _):
                p = pch[pl.ds(j * L, L)]
                t = tch[pl.ds(j * L, L)]
                kk = _ukey(p)
                d = p - t
                keybuf[pl.ds(c * CHUNK + j * L, L)] = kk
                d2buf[pl.ds(c * CHUNK + j * L, L)] = d * d
                bin8 = jnp.right_shift(kk, 24) & 255
                plsc.addupdate_scatter(hist, [lane * NBINS + bin8], ones_i)
                return 0

            lax.fori_loop(0, VPC, vec_a, 0)
            return 0

        lax.fori_loop(0, NCHUNKS, chunk_a, 0)

        m, kneed, accum = jnp.int32(NCOLS), jnp.int32(TOPK), jnp.zeros(
            (L,), jnp.float32)
        m, kneed, accum = radix_round(24, m, kneed, accum, build_hist=False)
        for shift in (16, 8, 0):
            m, kneed, accum = radix_round(shift, m, kneed, accum,
                                          build_hist=True)

        def fstep(j, acc):
            dd = d2buf[pl.ds(j * L, L)]
            sel = (j * L + lane) < kneed
            return acc + jnp.where(sel, dd, 0.0)

        accum = lax.fori_loop(0, (kneed + L - 1) // L, fstep, accum)
        return total + jnp.sum(accum)

    total = lax.fori_loop(0, ROWS_PER_W, row_fn, jnp.float32(0.0))
    outv[...] = jnp.where(lane == 0, total * (1.0 / (TOPK * NROWS)), 0.0)
    pltpu.sync_copy(outv, out_hbm.at[wid])


@jax.jit
def _topk_mse(pred_flat, true_flat):
    mesh = plsc.VectorSubcoreMesh(
        core_axis_name="c", subcore_axis_name="s", num_cores=NC,
        num_subcores=NS,
    )
    return pl.kernel(
        _body,
        out_type=jax.ShapeDtypeStruct((NW, L), jnp.float32),
        mesh=mesh,
        compiler_params=pltpu.CompilerParams(needs_layout_passes=False),
        scratch_types=[
            pltpu.VMEM((CHUNK,), jnp.float32),
            pltpu.VMEM((CHUNK,), jnp.float32),
            pltpu.VMEM((NBINS * L,), jnp.int32),
            pltpu.VMEM((NCOLS + L,), jnp.int32),
            pltpu.VMEM((NCOLS + L,), jnp.float32),
            pltpu.VMEM((L,), jnp.float32),
        ],
    )(pred_flat, true_flat)


def kernel(transformation_matrices, pred, true):
    del transformation_matrices
    pred = jnp.squeeze(pred)
    true = jnp.squeeze(true)
    partials = _topk_mse(pred.reshape(-1), true.reshape(-1))
    return jnp.sum(partials)
